# baseline (speedup 1.0000x reference)
import functools

import jax
import jax.numpy as jnp
from jax import lax
from jax.experimental import pallas as pl
from jax.experimental.pallas import tpu as pltpu
from jax.experimental.pallas import tpu_sc as plsc

VOCAB = 1000000
EMBED_DIM = 16
BATCH = 16384
HIST = 200

_PACK = 128 // EMBED_DIM
_ROWS = VOCAB // _PACK
_BLK = 1000
_GRID = _ROWS // _BLK

_NC = 2
_NS = 16
_NW = _NC * _NS
_RB = 16
_NB = BATCH // _RB
_BPW = _NB // _NW
_IDX_ROWS = HIST * _RB // 128


_SBLK = 4096
_SGRID = -(-VOCAB // _SBLK)


def _score_body(m_ref, s_ref, b_ref, o_ref):
    o_ref[...] = (
        jnp.dot(m_ref[...], s_ref[...], preferred_element_type=jnp.float32)
        + b_ref[0, 0]
    )


_score = pl.pallas_call(
    _score_body,
    grid=(_SGRID,),
    in_specs=[
        pl.BlockSpec((_SBLK, EMBED_DIM), lambda i: (i, 0)),
        pl.BlockSpec((EMBED_DIM, 1), lambda i: (0, 0)),
        pl.BlockSpec(memory_space=pltpu.SMEM),
    ],
    out_specs=pl.BlockSpec((_SBLK, 1), lambda i: (i, 0)),
    out_shape=jax.ShapeDtypeStruct((_SGRID * _SBLK, 1), jnp.float32),
)


_CHUNK = _IDX_ROWS * 128
_NBUF = 2
_J = HIST // 16
_TAIL = HIST - _J * 16


def _pool_body(xr_hbm, s_hbm, out_hbm, idx_v, g0, g1, out_v, sem0, sem1):
    g = [g0, g1]
    sem = [sem0, sem1]
    wid = lax.axis_index("s") * _NC + lax.axis_index("c")
    base = wid * _BPW

    pltpu.sync_copy(
        xr_hbm.at[pl.ds(base * _IDX_ROWS, _BPW * _IDX_ROWS), :], idx_v
    )

    def fire(blk, k):
        row0 = blk * _IDX_ROWS
        for i in range(_IDX_ROWS):
            pltpu.async_copy(
                s_hbm.at[idx_v.at[row0 + i]],
                g[k].at[pl.ds(i * 128, 128)],
                sem[k],
            )

    def wait_g(k):
        pltpu.make_async_copy(
            s_hbm.at[pl.ds(0, _CHUNK)], g[k].at[pl.ds(0, _CHUNK)], sem[k]
        ).wait()

    for k in range(_NBUF):
        fire(k, k)

    def body(it, carry):
        blk0 = it * _NBUF
        for k in range(_NBUF):
            blk = blk0 + k
            wait_g(k)
            acc = jnp.zeros((16,), jnp.float32)
            for i in range(_IDX_ROWS):
                for c in range(_PACK):
                    acc = acc + g[k][pl.ds(i * 128 + c * 16, 16)]
            m = acc * (1.0 / HIST)
            y = 1.0 / (1.0 + jnp.exp(-m))
            y = (y * 100.0 + 0.5).astype(jnp.int32).astype(jnp.float32) / 100.0
            out_v[pl.ds(blk * _RB, _RB)] = y
            nxt = blk + _NBUF

            @pl.when(nxt < _BPW)
            def _():
                fire(nxt, k)

        return carry

    lax.fori_loop(0, _BPW // _NBUF, body, 0)
    pltpu.sync_copy(out_v, out_hbm.at[pl.ds(base * _RB, _BPW * _RB)])


@functools.cache
def _pool():
    return pl.kernel(
        _pool_body,
        out_type=jax.ShapeDtypeStruct((BATCH,), jnp.float32),
        mesh=plsc.VectorSubcoreMesh(
            core_axis_name="c", subcore_axis_name="s",
            num_cores=_NC, num_subcores=_NS,
        ),
        scratch_types=[
            pltpu.VMEM((_BPW * _IDX_ROWS, 128), jnp.int32),
            pltpu.VMEM((_CHUNK,), jnp.float32),
            pltpu.VMEM((_CHUNK,), jnp.float32),
            pltpu.VMEM((_BPW * _RB,), jnp.float32),
            pltpu.SemaphoreType.DMA,
            pltpu.SemaphoreType.DMA,
        ],
    )


def kernel(x, table, W, b):
    xr = (
        x.astype(jnp.int32)
        .reshape(_NB, _RB, HIST)
        .transpose(0, 2, 1)
        .reshape(_NB * _IDX_ROWS, 128)
    )
    scores = _score(table, W.reshape(EMBED_DIM, 1), b.reshape(1, 1))
    y = _pool()(xr, scores.reshape(_SGRID * _SBLK))
    return y.reshape(BATCH, 1)


# speedup vs baseline: 1.0453x; 1.0453x over previous
import functools

import jax
import jax.numpy as jnp
from jax import lax
from jax.experimental import pallas as pl
from jax.experimental.pallas import tpu as pltpu
from jax.experimental.pallas import tpu_sc as plsc

VOCAB = 1000000
EMBED_DIM = 16
BATCH = 16384
HIST = 200

_PACK = 128 // EMBED_DIM
_ROWS = VOCAB // _PACK
_BLK = 1000
_GRID = _ROWS // _BLK

_NC = 2
_NS = 16
_NW = _NC * _NS
_RB = 16
_NB = BATCH // _RB
_BPW = _NB // _NW
_IDX_ROWS = HIST * _RB // 128


_SBLK = 8192
_SGRID = -(-VOCAB // _SBLK)


def _score_body(m_ref, w_ref, b_ref, o_ref):
    o_ref[...] = (
        jnp.sum(m_ref[...] * w_ref[...], axis=1, keepdims=True) + b_ref[0, 0]
    )


_score = pl.pallas_call(
    _score_body,
    grid=(_SGRID,),
    in_specs=[
        pl.BlockSpec((_SBLK, EMBED_DIM), lambda i: (i, 0)),
        pl.BlockSpec((1, EMBED_DIM), lambda i: (0, 0)),
        pl.BlockSpec(memory_space=pltpu.SMEM),
    ],
    out_specs=pl.BlockSpec((_SBLK, 1), lambda i: (i, 0)),
    out_shape=jax.ShapeDtypeStruct((_SGRID * _SBLK, 1), jnp.float32),
)


_CHUNK = _IDX_ROWS * 128
_NBUF = 2
_J = HIST // 16
_TAIL = HIST - _J * 16


def _pool_body(xr_hbm, s_hbm, out_hbm, idx_v, g0, g1, out_v, sem0, sem1):
    g = [g0, g1]
    sem = [sem0, sem1]
    wid = lax.axis_index("s") * _NC + lax.axis_index("c")
    base = wid * _BPW

    pltpu.sync_copy(
        xr_hbm.at[pl.ds(base * _IDX_ROWS, _BPW * _IDX_ROWS), :], idx_v
    )

    def fire(blk, k):
        row0 = blk * _IDX_ROWS
        for i in range(_IDX_ROWS):
            pltpu.async_copy(
                s_hbm.at[idx_v.at[row0 + i]],
                g[k].at[pl.ds(i * 128, 128)],
                sem[k],
            )

    def wait_g(k):
        pltpu.make_async_copy(
            s_hbm.at[pl.ds(0, _CHUNK)], g[k].at[pl.ds(0, _CHUNK)], sem[k]
        ).wait()

    for k in range(_NBUF):
        fire(k, k)

    def body(it, carry):
        blk0 = it * _NBUF
        for k in range(_NBUF):
            blk = blk0 + k
            wait_g(k)
            acc = jnp.zeros((16,), jnp.float32)
            for i in range(_IDX_ROWS):
                for c in range(_PACK):
                    acc = acc + g[k][pl.ds(i * 128 + c * 16, 16)]
            m = acc * (1.0 / HIST)
            y = 1.0 / (1.0 + jnp.exp(-m))
            y = (y * 100.0 + 0.5).astype(jnp.int32).astype(jnp.float32) / 100.0
            out_v[pl.ds(blk * _RB, _RB)] = y
            nxt = blk + _NBUF

            @pl.when(nxt < _BPW)
            def _():
                fire(nxt, k)

        return carry

    lax.fori_loop(0, _BPW // _NBUF, body, 0)
    pltpu.sync_copy(out_v, out_hbm.at[pl.ds(base * _RB, _BPW * _RB)])


@functools.cache
def _pool():
    return pl.kernel(
        _pool_body,
        out_type=jax.ShapeDtypeStruct((BATCH,), jnp.float32),
        mesh=plsc.VectorSubcoreMesh(
            core_axis_name="c", subcore_axis_name="s",
            num_cores=_NC, num_subcores=_NS,
        ),
        scratch_types=[
            pltpu.VMEM((_BPW * _IDX_ROWS, 128), jnp.int32),
            pltpu.VMEM((_CHUNK,), jnp.float32),
            pltpu.VMEM((_CHUNK,), jnp.float32),
            pltpu.VMEM((_BPW * _RB,), jnp.float32),
            pltpu.SemaphoreType.DMA,
            pltpu.SemaphoreType.DMA,
        ],
    )


def kernel(x, table, W, b):
    xr = (
        x.astype(jnp.int32)
        .reshape(_NB, _RB, HIST)
        .transpose(0, 2, 1)
        .reshape(_NB * _IDX_ROWS, 128)
    )
    scores = _score(table, W.reshape(1, EMBED_DIM), b.reshape(1, 1))
    y = _pool()(xr, scores.reshape(_SGRID * _SBLK))
    return y.reshape(BATCH, 1)


# speedup vs baseline: 1.3805x; 1.3207x over previous
import functools

import jax
import jax.numpy as jnp
from jax import lax
from jax.experimental import pallas as pl
from jax.experimental.pallas import tpu as pltpu
from jax.experimental.pallas import tpu_sc as plsc

VOCAB = 1000000
EMBED_DIM = 16
BATCH = 16384
HIST = 200

_PACK = 128 // EMBED_DIM
_ROWS = VOCAB // _PACK
_BLK = 5000
_GRID = _ROWS // _BLK

_NC = 2
_NS = 16
_NW = _NC * _NS
_RB = 16
_NB = BATCH // _RB
_BPW = _NB // _NW
_IDX_ROWS = HIST * _RB // 128


def _score_body(m_ref, s_ref, b_ref, o_ref):
    o_ref[...] = (
        jnp.dot(m_ref[...], s_ref[...], preferred_element_type=jnp.float32)
        + b_ref[0, 0]
    )


_score = pl.pallas_call(
    _score_body,
    grid=(_GRID,),
    in_specs=[
        pl.BlockSpec((_BLK, 128), lambda i: (i, 0)),
        pl.BlockSpec((128, _PACK), lambda i: (0, 0)),
        pl.BlockSpec(memory_space=pltpu.SMEM),
    ],
    out_specs=pl.BlockSpec((_BLK, _PACK), lambda i: (i, 0)),
    out_shape=jax.ShapeDtypeStruct((_ROWS, _PACK), jnp.float32),
)


_CHUNK = _IDX_ROWS * 128
_NBUF = 2
_J = HIST // 16
_TAIL = HIST - _J * 16


def _pool_body(xr_hbm, s_hbm, out_hbm, idx_v, g0, g1, out_v, sem0, sem1):
    g = [g0, g1]
    sem = [sem0, sem1]
    wid = lax.axis_index("s") * _NC + lax.axis_index("c")
    base = wid * _BPW

    pltpu.sync_copy(
        xr_hbm.at[pl.ds(base * _IDX_ROWS, _BPW * _IDX_ROWS), :], idx_v
    )

    def fire(blk, k):
        row0 = blk * _IDX_ROWS
        for i in range(_IDX_ROWS):
            pltpu.async_copy(
                s_hbm.at[idx_v.at[row0 + i]],
                g[k].at[pl.ds(i * 128, 128)],
                sem[k],
            )

    def wait_g(k):
        pltpu.make_async_copy(
            s_hbm.at[pl.ds(0, _CHUNK)], g[k].at[pl.ds(0, _CHUNK)], sem[k]
        ).wait()

    for k in range(_NBUF):
        fire(k, k)

    def body(it, carry):
        blk0 = it * _NBUF
        for k in range(_NBUF):
            blk = blk0 + k
            wait_g(k)
            acc = jnp.zeros((16,), jnp.float32)
            for i in range(_IDX_ROWS):
                for c in range(_PACK):
                    acc = acc + g[k][pl.ds(i * 128 + c * 16, 16)]
            m = acc * (1.0 / HIST)
            y = 1.0 / (1.0 + jnp.exp(-m))
            y = (y * 100.0 + 0.5).astype(jnp.int32).astype(jnp.float32) / 100.0
            out_v[pl.ds(blk * _RB, _RB)] = y
            nxt = blk + _NBUF

            @pl.when(nxt < _BPW)
            def _():
                fire(nxt, k)

        return carry

    lax.fori_loop(0, _BPW // _NBUF, body, 0)
    pltpu.sync_copy(out_v, out_hbm.at[pl.ds(base * _RB, _BPW * _RB)])


@functools.cache
def _pool():
    return pl.kernel(
        _pool_body,
        out_type=jax.ShapeDtypeStruct((BATCH,), jnp.float32),
        mesh=plsc.VectorSubcoreMesh(
            core_axis_name="c", subcore_axis_name="s",
            num_cores=_NC, num_subcores=_NS,
        ),
        scratch_types=[
            pltpu.VMEM((_BPW * _IDX_ROWS, 128), jnp.int32),
            pltpu.VMEM((_CHUNK,), jnp.float32),
            pltpu.VMEM((_CHUNK,), jnp.float32),
            pltpu.VMEM((_BPW * _RB,), jnp.float32),
            pltpu.SemaphoreType.DMA,
            pltpu.SemaphoreType.DMA,
        ],
    )


def kernel(x, table, W, b):
    xr = (
        x.astype(jnp.int32)
        .reshape(_NB, _RB, HIST)
        .transpose(0, 2, 1)
        .reshape(_NB * _IDX_ROWS, 128)
    )
    sel = jnp.repeat(jnp.eye(_PACK, dtype=jnp.float32), EMBED_DIM, axis=0)
    sel = sel * jnp.tile(W.reshape(EMBED_DIM), _PACK)[:, None]
    scores = _score(table.reshape(_ROWS, 128), sel, b.reshape(1, 1))
    y = _pool()(xr, scores.reshape(VOCAB))
    return y.reshape(BATCH, 1)


# speedup vs baseline: 1.6262x; 1.1780x over previous
import functools

import jax
import jax.numpy as jnp
from jax import lax
from jax.experimental import pallas as pl
from jax.experimental.pallas import tpu as pltpu
from jax.experimental.pallas import tpu_sc as plsc

VOCAB = 1000000
EMBED_DIM = 16
BATCH = 16384
HIST = 200

_PACK = 128 // EMBED_DIM
_ROWS = VOCAB // _PACK
_BLK = 5000
_GRID = _ROWS // _BLK

_NC = 2
_NS = 16
_NW = _NC * _NS
_RB = 16
_NB = BATCH // _RB
_BPW = _NB // _NW
_IDX_ROWS = HIST * _RB // 128


def _score_body(m_ref, s_ref, b_ref, o_ref):
    o_ref[...] = (
        jnp.dot(m_ref[...], s_ref[...], preferred_element_type=jnp.float32)
        + b_ref[0, 0]
    )


_score = pl.pallas_call(
    _score_body,
    grid=(_GRID,),
    in_specs=[
        pl.BlockSpec((_BLK, 128), lambda i: (i, 0)),
        pl.BlockSpec((128, _PACK), lambda i: (0, 0)),
        pl.BlockSpec(memory_space=pltpu.SMEM),
    ],
    out_specs=pl.BlockSpec((_BLK, _PACK), lambda i: (i, 0)),
    out_shape=jax.ShapeDtypeStruct((_ROWS, _PACK), jnp.float32),
)


_CHUNK = _IDX_ROWS * 128
_NBUF = 2
_J = HIST // 16
_TAIL = HIST - _J * 16


def _pool_body(
    xr_hbm, s_hbm, out_hbm, i0, i1, g0, g1, s_sh, out_v, si0, si1, sg0, sg1
):
    idx = [i0, i1]
    g = [g0, g1]
    semi = [si0, si1]
    semg = [sg0, sg1]
    sid = lax.axis_index("s")
    wid = sid * _NC + lax.axis_index("c")
    base = wid * _BPW

    @pl.when(sid == 0)
    def _():
        pltpu.sync_copy(s_hbm, s_sh)

    plsc.subcore_barrier()

    def fire_idx(blk, k):
        pltpu.async_copy(xr_hbm.at[base + blk], idx[k], semi[k])

    def wait_idx(k):
        pltpu.make_async_copy(xr_hbm.at[0], idx[k], semi[k]).wait()

    def fire_g(blk, k):
        for i in range(_IDX_ROWS):
            pltpu.async_copy(
                s_sh.at[idx[k].at[i]],
                g[k].at[pl.ds(i * 128, 128)],
                semg[k],
            )

    def wait_g(k):
        pltpu.make_async_copy(
            s_hbm.at[pl.ds(0, _CHUNK)], g[k].at[pl.ds(0, _CHUNK)], semg[k]
        ).wait()

    fire_idx(0, 0)
    fire_idx(1, 1)
    wait_idx(0)
    fire_g(0, 0)

    def body(it, carry):
        for k in range(2):
            blk = it * 2 + k
            nk = (k + 1) % 2

            @pl.when(blk + 1 < _BPW)
            def _():
                wait_idx(nk)
                fire_g(blk + 1, nk)

            wait_g(k)
            acc = jnp.zeros((16,), jnp.float32)
            for i in range(_IDX_ROWS):
                for c in range(_PACK):
                    acc = acc + g[k][pl.ds(i * 128 + c * 16, 16)]
            m = acc * (1.0 / HIST)
            y = 1.0 / (1.0 + jnp.exp(-m))
            y = (y * 100.0 + 0.5).astype(jnp.int32).astype(jnp.float32) / 100.0
            out_v[pl.ds(blk * _RB, _RB)] = y

            @pl.when(blk + 2 < _BPW)
            def _():
                fire_idx(blk + 2, k)

        return carry

    lax.fori_loop(0, _BPW // 2, body, 0)
    pltpu.sync_copy(out_v, out_hbm.at[pl.ds(base * _RB, _BPW * _RB)])


@functools.cache
def _pool():
    return pl.kernel(
        _pool_body,
        out_type=jax.ShapeDtypeStruct((BATCH,), jnp.float32),
        mesh=plsc.VectorSubcoreMesh(
            core_axis_name="c", subcore_axis_name="s",
            num_cores=_NC, num_subcores=_NS,
        ),
        scratch_types=[
            pltpu.VMEM((_IDX_ROWS, 128), jnp.int32),
            pltpu.VMEM((_IDX_ROWS, 128), jnp.int32),
            pltpu.VMEM((_CHUNK,), jnp.float32),
            pltpu.VMEM((_CHUNK,), jnp.float32),
            pltpu.VMEM_SHARED((VOCAB,), jnp.float32),
            pltpu.VMEM((_BPW * _RB,), jnp.float32),
            pltpu.SemaphoreType.DMA,
            pltpu.SemaphoreType.DMA,
            pltpu.SemaphoreType.DMA,
            pltpu.SemaphoreType.DMA,
        ],
    )


def kernel(x, table, W, b):
    xr = (
        x.astype(jnp.int32)
        .reshape(_NB, _RB, HIST)
        .transpose(0, 2, 1)
        .reshape(_NB, _IDX_ROWS, 128)
    )
    sel = jnp.repeat(jnp.eye(_PACK, dtype=jnp.float32), EMBED_DIM, axis=0)
    sel = sel * jnp.tile(W.reshape(EMBED_DIM), _PACK)[:, None]
    scores = _score(table.reshape(_ROWS, 128), sel, b.reshape(1, 1))
    y = _pool()(xr, scores.reshape(VOCAB))
    return y.reshape(BATCH, 1)
